# baseline (speedup 1.0000x reference)
import numpy as np
import jax
import jax.numpy as jnp
from jax.experimental import pallas as pl
from jax.experimental.pallas import tpu as pltpu
from jax.experimental.pallas import tpu_sc as plsc

_VOCAB = 100000
_EMB = 128
_NPOS = 200
_HIDDEN = 50
_XROW = 400
_XCOL = 64
_WLANE = _XCOL * _HIDDEN
_VBLK = 12800
_NBLK = 8
_GW = 128
_NPAD = 256

_RT = (np.arange(_WLANE)[None, :] // _HIDDEN
       == np.arange(_XCOL)[:, None]).astype(np.float32)


def _sc_gather(emb, idx2d):
    mesh = plsc.VectorSubcoreMesh(core_axis_name="core",
                                  subcore_axis_name="subcore")

    @pl.kernel(out_type=jax.ShapeDtypeStruct((_NPAD, _EMB), jnp.float32),
               mesh=mesh)
    def gather_kernel(emb_hbm, i_hbm, o_hbm):
        def body(i_vmem, o_vmem):
            pltpu.sync_copy(emb_hbm.at[i_vmem.at[0]], o_vmem)

        pltpu.emit_pipeline(
            body,
            grid=(_NPAD // _GW,),
            in_specs=[pl.BlockSpec((1, _GW), lambda i: (0, i))],
            out_specs=[pl.BlockSpec((_GW, _EMB), lambda i: (i, 0))],
            core_axis_name=("core", "subcore"),
            dimension_semantics=(pltpu.PARALLEL,),
        )(i_hbm, o_hbm)

    return gather_kernel(emb, idx2d)


def _lin1_body(x_ref, rt_ref, v_ref, b1_ref, o_ref):
    x64 = x_ref[0 : _XROW, :]
    xr = jnp.dot(x64, rt_ref[...], preferred_element_type=jnp.float32)
    s = jnp.sum(xr * v_ref[...], axis=0, keepdims=True)
    for r in (1600, 800, 400, 200, 100, 50):
        s = s + pltpu.roll(s, _WLANE - r, axis=1)
    h = jnp.maximum(s[0:1, 0:_HIDDEN] + b1_ref[...], 0.0)
    eye = (jax.lax.broadcasted_iota(jnp.int32, (_HIDDEN, _HIDDEN), 0)
           == jax.lax.broadcasted_iota(jnp.int32, (_HIDDEN, _HIDDEN), 1)
           ).astype(jnp.float32)
    hcol = jnp.sum(
        jnp.dot(jnp.ones((_HIDDEN, 1), jnp.float32), h,
                preferred_element_type=jnp.float32) * eye,
        axis=1, keepdims=True)
    o_ref[...] = hcol


def _lin1(x512, W1, b1):
    v = W1.reshape(_XROW, _WLANE)
    return pl.pallas_call(
        _lin1_body,
        in_specs=[
            pl.BlockSpec((2 * _NPAD, _XCOL), lambda: (0, 0)),
            pl.BlockSpec((_XCOL, _WLANE), lambda: (0, 0)),
            pl.BlockSpec((_XROW, _WLANE), lambda: (0, 0)),
            pl.BlockSpec((1, _HIDDEN), lambda: (0, 0)),
        ],
        out_specs=pl.BlockSpec((_HIDDEN, 1), lambda: (0, 0)),
        out_shape=jax.ShapeDtypeStruct((_HIDDEN, 1), jnp.float32),
    )(x512, jnp.asarray(_RT), v, b1.reshape(1, _HIDDEN))


def _lin2_body(hc_ref, w2_ref, b2_ref, o_ref, acc_ref, m_ref, l_ref):
    k = pl.program_id(0)

    @pl.when(k < _NBLK)
    def _():
        zp = (jnp.sum(w2_ref[...] * hc_ref[...], axis=0, keepdims=True)
              + b2_ref[...])
        lane = jax.lax.broadcasted_iota(jnp.int32, (1, _VBLK), 1)
        valid = (k * _VBLK + lane) < _VOCAB
        zm = jnp.where(valid, zp, -jnp.inf)
        mk = jnp.max(zm)
        lk = jnp.sum(jnp.where(valid, jnp.exp(zm - mk), 0.0))
        for j in range(_NBLK):
            @pl.when(k == j)
            def _():
                acc_ref[0:1, pl.ds(j * _VBLK, _VBLK)] = zp

        @pl.when(k == 0)
        def _():
            m_ref[0] = mk
            l_ref[0] = lk

        @pl.when(k > 0)
        def _():
            m_old = m_ref[0]
            m_new = jnp.maximum(m_old, mk)
            l_ref[0] = (l_ref[0] * jnp.exp(m_old - m_new)
                        + lk * jnp.exp(mk - m_new))
            m_ref[0] = m_new

    @pl.when(k == _NBLK)
    def _():
        lse = jnp.log(l_ref[0]) + m_ref[0]
        o_ref[...] = acc_ref[0:1, pl.ds(0, _VOCAB)] - lse


def _lin2_logsoftmax(h_col, W2, b2):
    grid_spec = pltpu.PrefetchScalarGridSpec(
        num_scalar_prefetch=0,
        grid=(_NBLK + 1,),
        in_specs=[
            pl.BlockSpec((_HIDDEN, 1), lambda k: (0, 0)),
            pl.BlockSpec((_HIDDEN, _VBLK),
                         lambda k: (0, jnp.minimum(k, _NBLK - 1))),
            pl.BlockSpec((1, _VBLK),
                         lambda k: (0, jnp.minimum(k, _NBLK - 1))),
        ],
        out_specs=pl.BlockSpec((1, _VOCAB), lambda k: (0, 0)),
        scratch_shapes=[
            pltpu.VMEM((1, _NBLK * _VBLK), jnp.float32),
            pltpu.SMEM((1,), jnp.float32),
            pltpu.SMEM((1,), jnp.float32),
        ],
    )
    return pl.pallas_call(
        _lin2_body,
        grid_spec=grid_spec,
        out_shape=jax.ShapeDtypeStruct((1, _VOCAB), jnp.float32),
    )(h_col, W2, b2.reshape(1, _VOCAB))


def kernel(inp, emb, W1, b1, W2, b2):
    idx = inp.astype(jnp.int32)
    idx2d = jnp.zeros((1, _NPAD), jnp.int32).at[0, :_NPOS].set(idx)
    xg = _sc_gather(emb, idx2d)
    x512 = xg.reshape(2 * _NPAD, _XCOL)
    h_col = _lin1(x512, W1, b1)
    return _lin2_logsoftmax(h_col, W2, b2)


# speedup vs baseline: 1.2236x; 1.2236x over previous
import jax
import jax.numpy as jnp
from jax.experimental import pallas as pl
from jax.experimental.pallas import tpu as pltpu
from jax.experimental.pallas import tpu_sc as plsc

_VOCAB = 100000
_EMB = 128
_NPOS = 200
_HIDDEN = 50
_XLEN = _NPOS * _EMB
_NSTREAM = 4
_W1CHUNK = _XLEN // _NSTREAM
_VBLK = 12800
_NBLK = 8
_GW = 128
_NPAD = 256


def _sc_gather(emb, idx2d):
    mesh = plsc.VectorSubcoreMesh(core_axis_name="core",
                                  subcore_axis_name="subcore")

    @pl.kernel(out_type=jax.ShapeDtypeStruct((_NPAD, _EMB), jnp.float32),
               mesh=mesh)
    def gather_kernel(emb_hbm, i_hbm, o_hbm):
        def body(i_vmem, o_vmem):
            pltpu.sync_copy(emb_hbm.at[i_vmem.at[0]], o_vmem)

        pltpu.emit_pipeline(
            body,
            grid=(_NPAD // _GW,),
            in_specs=[pl.BlockSpec((1, _GW), lambda i: (0, i))],
            out_specs=[pl.BlockSpec((_GW, _EMB), lambda i: (i, 0))],
            core_axis_name=("core", "subcore"),
            dimension_semantics=(pltpu.PARALLEL,),
        )(i_hbm, o_hbm)

    return gather_kernel(emb, idx2d)


def _lin1_body(x_ref, w1_hbm, b1_ref, o_ref, w1buf, sems):
    for q in range(_NSTREAM):
        pltpu.make_async_copy(
            w1_hbm.at[pl.ds(q * _W1CHUNK, _W1CHUNK), :],
            w1buf.at[pl.ds(q * _W1CHUNK, _W1CHUNK), :],
            sems.at[q],
        ).start()
    for q in range(_NSTREAM):
        pltpu.make_async_copy(
            w1_hbm.at[pl.ds(q * _W1CHUNK, _W1CHUNK), :],
            w1buf.at[pl.ds(q * _W1CHUNK, _W1CHUNK), :],
            sems.at[q],
        ).wait()
    x = x_ref[0:1, 0:_XLEN]
    h = jnp.maximum(
        jnp.dot(x, w1buf[...], preferred_element_type=jnp.float32)
        + b1_ref[...][None, :], 0.0)
    eye = (jax.lax.broadcasted_iota(jnp.int32, (_HIDDEN, _HIDDEN), 0)
           == jax.lax.broadcasted_iota(jnp.int32, (_HIDDEN, _HIDDEN), 1)
           ).astype(jnp.float32)
    o_ref[...] = jnp.sum(
        jnp.dot(jnp.ones((_HIDDEN, 1), jnp.float32), h,
                preferred_element_type=jnp.float32) * eye,
        axis=1, keepdims=True)


def _lin1(xflat, W1, b1):
    return pl.pallas_call(
        _lin1_body,
        in_specs=[
            pl.BlockSpec((1, xflat.shape[1]), lambda: (0, 0)),
            pl.BlockSpec(memory_space=pltpu.MemorySpace.HBM),
            pl.BlockSpec((_HIDDEN,), lambda: (0,)),
        ],
        out_specs=pl.BlockSpec((_HIDDEN, 1), lambda: (0, 0)),
        out_shape=jax.ShapeDtypeStruct((_HIDDEN, 1), jnp.float32),
        scratch_shapes=[
            pltpu.VMEM((_XLEN, _HIDDEN), jnp.float32),
            pltpu.SemaphoreType.DMA((_NSTREAM,)),
        ],
    )(xflat, W1, b1)


def _lin2_body(hc_ref, w2_ref, b2_ref, o_ref, acc_ref, m_ref, l_ref):
    k = pl.program_id(0)

    @pl.when(k < _NBLK)
    def _():
        zp = (jnp.sum(w2_ref[...] * hc_ref[...], axis=0, keepdims=True)
              + b2_ref[...])
        lane = jax.lax.broadcasted_iota(jnp.int32, (1, _VBLK), 1)
        valid = (k * _VBLK + lane) < _VOCAB
        zm = jnp.where(valid, zp, -jnp.inf)
        mk = jnp.max(zm)
        lk = jnp.sum(jnp.where(valid, jnp.exp(zm - mk), 0.0))
        for j in range(_NBLK):
            @pl.when(k == j)
            def _():
                acc_ref[0:1, pl.ds(j * _VBLK, _VBLK)] = zp

        @pl.when(k == 0)
        def _():
            m_ref[0] = mk
            l_ref[0] = lk

        @pl.when(k > 0)
        def _():
            m_old = m_ref[0]
            m_new = jnp.maximum(m_old, mk)
            l_ref[0] = (l_ref[0] * jnp.exp(m_old - m_new)
                        + lk * jnp.exp(mk - m_new))
            m_ref[0] = m_new

    @pl.when(k == _NBLK)
    def _():
        lse = jnp.log(l_ref[0]) + m_ref[0]
        o_ref[...] = acc_ref[0:1, pl.ds(0, _VOCAB)] - lse


def _lin2_logsoftmax(h_col, W2, b2):
    grid_spec = pltpu.PrefetchScalarGridSpec(
        num_scalar_prefetch=0,
        grid=(_NBLK + 1,),
        in_specs=[
            pl.BlockSpec((_HIDDEN, 1), lambda k: (0, 0)),
            pl.BlockSpec((_HIDDEN, _VBLK),
                         lambda k: (0, jnp.minimum(k, _NBLK - 1))),
            pl.BlockSpec((1, _VBLK),
                         lambda k: (0, jnp.minimum(k, _NBLK - 1))),
        ],
        out_specs=pl.BlockSpec((1, _VOCAB), lambda k: (0, 0)),
        scratch_shapes=[
            pltpu.VMEM((1, _NBLK * _VBLK), jnp.float32),
            pltpu.SMEM((1,), jnp.float32),
            pltpu.SMEM((1,), jnp.float32),
        ],
    )
    return pl.pallas_call(
        _lin2_body,
        grid_spec=grid_spec,
        out_shape=jax.ShapeDtypeStruct((1, _VOCAB), jnp.float32),
    )(h_col, W2, b2.reshape(1, _VOCAB))


def kernel(inp, emb, W1, b1, W2, b2):
    idx = inp.astype(jnp.int32)
    idx2d = jnp.zeros((1, _NPAD), jnp.int32).at[0, :_NPOS].set(idx)
    xg = _sc_gather(emb, idx2d)
    xflat = xg.reshape(1, _NPAD * _EMB)
    h_col = _lin1(xflat, W1, b1)
    return _lin2_logsoftmax(h_col, W2, b2)


# speedup vs baseline: 1.7667x; 1.4438x over previous
import jax
import jax.numpy as jnp
from jax.experimental import pallas as pl
from jax.experimental.pallas import tpu as pltpu

_VOCAB = 100000
_EMB = 128
_NPOS = 200
_HIDDEN = 50
_XLEN = _NPOS * _EMB
_W1BLKS = 8
_W1ROWS = _XLEN // _W1BLKS
_VBLK = 12800
_NBLK = 8


def _lin1_body(idx_ref, emb_hbm, w1_ref, b1_ref, o_ref,
               xbuf, xrow, hacc, gsem):
    q = pl.program_id(0)

    @pl.when(q == 0)
    def _():
        def issue(j, _):
            pltpu.make_async_copy(
                emb_hbm.at[pl.ds(idx_ref[j], 1), :],
                xbuf.at[pl.ds(j, 1), :], gsem).start()
            return _

        def drain(j, _):
            pltpu.make_async_copy(
                emb_hbm.at[pl.ds(0, 1), :],
                xbuf.at[pl.ds(0, 1), :], gsem).wait()
            return _

        jax.lax.fori_loop(0, _NPOS, issue, 0)
        jax.lax.fori_loop(0, _NPOS, drain, 0)
        xrow[...] = jnp.reshape(xbuf[...], (1, _XLEN))

    xc = xrow[0:1, pl.ds(q * _W1ROWS, _W1ROWS)].astype(jnp.bfloat16)
    part = jnp.dot(xc, w1_ref[...].astype(jnp.bfloat16),
                   preferred_element_type=jnp.float32)

    @pl.when(q == 0)
    def _():
        hacc[...] = part

    @pl.when(q > 0)
    def _():
        hacc[...] += part

    @pl.when(q == _W1BLKS - 1)
    def _():
        h = jnp.maximum(hacc[...] + b1_ref[...][None, :], 0.0)
        eye = (jax.lax.broadcasted_iota(jnp.int32, (_HIDDEN, _HIDDEN), 0)
               == jax.lax.broadcasted_iota(jnp.int32, (_HIDDEN, _HIDDEN), 1)
               ).astype(jnp.float32)
        o_ref[...] = jnp.sum(
            jnp.dot(jnp.ones((_HIDDEN, 1), jnp.float32), h,
                    preferred_element_type=jnp.float32) * eye,
            axis=1, keepdims=True)


def _lin1(idx, emb, W1, b1):
    return pl.pallas_call(
        _lin1_body,
        grid=(_W1BLKS,),
        in_specs=[
            pl.BlockSpec(memory_space=pltpu.MemorySpace.SMEM),
            pl.BlockSpec(memory_space=pltpu.MemorySpace.HBM),
            pl.BlockSpec((_W1ROWS, _HIDDEN), lambda q: (q, 0)),
            pl.BlockSpec((_HIDDEN,), lambda q: (0,)),
        ],
        out_specs=pl.BlockSpec((_HIDDEN, 1), lambda q: (0, 0)),
        out_shape=jax.ShapeDtypeStruct((_HIDDEN, 1), jnp.float32),
        scratch_shapes=[
            pltpu.VMEM((_NPOS, _EMB), jnp.float32),
            pltpu.VMEM((1, _XLEN), jnp.float32),
            pltpu.VMEM((1, _HIDDEN), jnp.float32),
            pltpu.SemaphoreType.DMA,
        ],
    )(idx, emb, W1, b1)


def _lin2_body(hc_ref, w2_ref, b2_ref, o_ref, acc_ref, m_ref, l_ref):
    k = pl.program_id(0)

    @pl.when(k < _NBLK)
    def _():
        zp = (jnp.sum(w2_ref[...] * hc_ref[...], axis=0, keepdims=True)
              + b2_ref[...])
        lane = jax.lax.broadcasted_iota(jnp.int32, (1, _VBLK), 1)
        valid = (k * _VBLK + lane) < _VOCAB
        zm = jnp.where(valid, zp, -jnp.inf)
        mk = jnp.max(zm)
        lk = jnp.sum(jnp.where(valid, jnp.exp(zm - mk), 0.0))
        for j in range(_NBLK):
            @pl.when(k == j)
            def _():
                acc_ref[0:1, pl.ds(j * _VBLK, _VBLK)] = zp

        @pl.when(k == 0)
        def _():
            m_ref[0] = mk
            l_ref[0] = lk

        @pl.when(k > 0)
        def _():
            m_old = m_ref[0]
            m_new = jnp.maximum(m_old, mk)
            l_ref[0] = (l_ref[0] * jnp.exp(m_old - m_new)
                        + lk * jnp.exp(mk - m_new))
            m_ref[0] = m_new

    @pl.when(k == _NBLK)
    def _():
        lse = jnp.log(l_ref[0]) + m_ref[0]
        o_ref[...] = acc_ref[0:1, pl.ds(0, _VOCAB)] - lse


def _lin2_logsoftmax(h_col, W2, b2):
    grid_spec = pltpu.PrefetchScalarGridSpec(
        num_scalar_prefetch=0,
        grid=(_NBLK + 1,),
        in_specs=[
            pl.BlockSpec((_HIDDEN, 1), lambda k: (0, 0)),
            pl.BlockSpec((_HIDDEN, _VBLK),
                         lambda k: (0, jnp.minimum(k, _NBLK - 1))),
            pl.BlockSpec((1, _VBLK),
                         lambda k: (0, jnp.minimum(k, _NBLK - 1))),
        ],
        out_specs=pl.BlockSpec((1, _VOCAB), lambda k: (0, 0)),
        scratch_shapes=[
            pltpu.VMEM((1, _NBLK * _VBLK), jnp.float32),
            pltpu.SMEM((1,), jnp.float32),
            pltpu.SMEM((1,), jnp.float32),
        ],
    )
    return pl.pallas_call(
        _lin2_body,
        grid_spec=grid_spec,
        out_shape=jax.ShapeDtypeStruct((1, _VOCAB), jnp.float32),
    )(h_col, W2, b2.reshape(1, _VOCAB))


def kernel(inp, emb, W1, b1, W2, b2):
    idx = inp.astype(jnp.int32)
    h_col = _lin1(idx, emb, W1, b1)
    return _lin2_logsoftmax(h_col, W2, b2)


# speedup vs baseline: 1.8046x; 1.0215x over previous
import jax
import jax.numpy as jnp
from jax.experimental import pallas as pl
from jax.experimental.pallas import tpu as pltpu

_VOCAB = 100000
_EMB = 128
_NPOS = 200
_HIDDEN = 50
_XLEN = _NPOS * _EMB
_W1BLKS = 8
_W1ROWS = _XLEN // _W1BLKS
_VBLK = 12800
_NBLK = 8


def _lin1_body(idx_ref, emb_hbm, w1_ref, b1_ref, o_ref,
               xbuf, xrow, hacc, gsem):
    q = pl.program_id(0)

    @pl.when(q == 0)
    def _():
        for j in range(_NPOS):
            pltpu.make_async_copy(
                emb_hbm.at[pl.ds(idx_ref[j], 1), :],
                xbuf.at[pl.ds(j, 1), :], gsem).start()
        pltpu.make_async_copy(
            emb_hbm.at[pl.ds(0, _NPOS), :], xbuf, gsem).wait()
        xrow[...] = jnp.reshape(xbuf[...], (1, _XLEN))

    xc = xrow[0:1, pl.ds(q * _W1ROWS, _W1ROWS)].astype(jnp.bfloat16)
    part = jnp.dot(xc, w1_ref[...].astype(jnp.bfloat16),
                   preferred_element_type=jnp.float32)

    @pl.when(q == 0)
    def _():
        hacc[...] = part

    @pl.when(q > 0)
    def _():
        hacc[...] += part

    @pl.when(q == _W1BLKS - 1)
    def _():
        h = jnp.maximum(hacc[...] + b1_ref[...][None, :], 0.0)
        eye = (jax.lax.broadcasted_iota(jnp.int32, (_HIDDEN, _HIDDEN), 0)
               == jax.lax.broadcasted_iota(jnp.int32, (_HIDDEN, _HIDDEN), 1)
               ).astype(jnp.float32)
        o_ref[...] = jnp.sum(
            jnp.dot(jnp.ones((_HIDDEN, 1), jnp.float32), h,
                    preferred_element_type=jnp.float32) * eye,
            axis=1, keepdims=True)


def _lin1(idx, emb, W1, b1):
    return pl.pallas_call(
        _lin1_body,
        grid=(_W1BLKS,),
        in_specs=[
            pl.BlockSpec(memory_space=pltpu.MemorySpace.SMEM),
            pl.BlockSpec(memory_space=pltpu.MemorySpace.HBM),
            pl.BlockSpec((_W1ROWS, _HIDDEN), lambda q: (q, 0)),
            pl.BlockSpec((_HIDDEN,), lambda q: (0,)),
        ],
        out_specs=pl.BlockSpec((_HIDDEN, 1), lambda q: (0, 0)),
        out_shape=jax.ShapeDtypeStruct((_HIDDEN, 1), jnp.float32),
        scratch_shapes=[
            pltpu.VMEM((_NPOS, _EMB), jnp.float32),
            pltpu.VMEM((1, _XLEN), jnp.float32),
            pltpu.VMEM((1, _HIDDEN), jnp.float32),
            pltpu.SemaphoreType.DMA,
        ],
    )(idx, emb, W1, b1)


def _lin2_body(hc_ref, w2_ref, b2_ref, o_ref, acc_ref, m_ref, l_ref):
    k = pl.program_id(0)

    @pl.when(k < _NBLK)
    def _():
        zp = (jnp.sum(w2_ref[...] * hc_ref[...], axis=0, keepdims=True)
              + b2_ref[...])
        lane = jax.lax.broadcasted_iota(jnp.int32, (1, _VBLK), 1)
        valid = (k * _VBLK + lane) < _VOCAB
        zm = jnp.where(valid, zp, -jnp.inf)
        mk = jnp.max(zm)
        lk = jnp.sum(jnp.where(valid, jnp.exp(zm - mk), 0.0))
        for j in range(_NBLK):
            @pl.when(k == j)
            def _():
                acc_ref[0:1, pl.ds(j * _VBLK, _VBLK)] = zp

        @pl.when(k == 0)
        def _():
            m_ref[0] = mk
            l_ref[0] = lk

        @pl.when(k > 0)
        def _():
            m_old = m_ref[0]
            m_new = jnp.maximum(m_old, mk)
            l_ref[0] = (l_ref[0] * jnp.exp(m_old - m_new)
                        + lk * jnp.exp(mk - m_new))
            m_ref[0] = m_new

    @pl.when(k == _NBLK)
    def _():
        lse = jnp.log(l_ref[0]) + m_ref[0]
        o_ref[...] = acc_ref[0:1, pl.ds(0, _VOCAB)] - lse


def _lin2_logsoftmax(h_col, W2, b2):
    grid_spec = pltpu.PrefetchScalarGridSpec(
        num_scalar_prefetch=0,
        grid=(_NBLK + 1,),
        in_specs=[
            pl.BlockSpec((_HIDDEN, 1), lambda k: (0, 0)),
            pl.BlockSpec((_HIDDEN, _VBLK),
                         lambda k: (0, jnp.minimum(k, _NBLK - 1))),
            pl.BlockSpec((1, _VBLK),
                         lambda k: (0, jnp.minimum(k, _NBLK - 1))),
        ],
        out_specs=pl.BlockSpec((1, _VOCAB), lambda k: (0, 0)),
        scratch_shapes=[
            pltpu.VMEM((1, _NBLK * _VBLK), jnp.float32),
            pltpu.SMEM((1,), jnp.float32),
            pltpu.SMEM((1,), jnp.float32),
        ],
    )
    return pl.pallas_call(
        _lin2_body,
        grid_spec=grid_spec,
        out_shape=jax.ShapeDtypeStruct((1, _VOCAB), jnp.float32),
    )(h_col, W2, b2.reshape(1, _VOCAB))


def kernel(inp, emb, W1, b1, W2, b2):
    idx = inp.astype(jnp.int32)
    h_col = _lin1(idx, emb, W1, b1)
    return _lin2_logsoftmax(h_col, W2, b2)


# speedup vs baseline: 2.0483x; 1.1351x over previous
import jax
import jax.numpy as jnp
from jax.experimental import pallas as pl
from jax.experimental.pallas import tpu as pltpu

_VOCAB = 100000
_EMB = 128
_NPOS = 200
_HIDDEN = 50
_XLEN = _NPOS * _EMB
_W1BLKS = 4
_W1ROWS = _XLEN // _W1BLKS
_VBLK = 25600
_NBLK = 4


def _lin1_body(idx_ref, emb_hbm, w1_ref, b1_ref, o_ref,
               xbuf, xrow, hacc, gsem):
    q = pl.program_id(0)

    @pl.when(q == 0)
    def _():
        for j in range(_NPOS):
            pltpu.make_async_copy(
                emb_hbm.at[pl.ds(idx_ref[j], 1), :],
                xbuf.at[pl.ds(j, 1), :], gsem).start()
        pltpu.make_async_copy(
            emb_hbm.at[pl.ds(0, _NPOS), :], xbuf, gsem).wait()
        xrow[...] = jnp.reshape(xbuf[...], (1, _XLEN))

    xc = xrow[0:1, pl.ds(q * _W1ROWS, _W1ROWS)].astype(jnp.bfloat16)
    part = jnp.dot(xc, w1_ref[...].astype(jnp.bfloat16),
                   preferred_element_type=jnp.float32)

    @pl.when(q == 0)
    def _():
        hacc[...] = part

    @pl.when(q > 0)
    def _():
        hacc[...] += part

    @pl.when(q == _W1BLKS - 1)
    def _():
        h = jnp.maximum(hacc[...] + b1_ref[...][None, :], 0.0)
        eye = (jax.lax.broadcasted_iota(jnp.int32, (_HIDDEN, _HIDDEN), 0)
               == jax.lax.broadcasted_iota(jnp.int32, (_HIDDEN, _HIDDEN), 1)
               ).astype(jnp.float32)
        o_ref[...] = jnp.sum(
            jnp.dot(jnp.ones((_HIDDEN, 1), jnp.float32), h,
                    preferred_element_type=jnp.float32) * eye,
            axis=1, keepdims=True)


def _lin1(idx, emb, W1, b1):
    return pl.pallas_call(
        _lin1_body,
        grid=(_W1BLKS,),
        in_specs=[
            pl.BlockSpec(memory_space=pltpu.MemorySpace.SMEM),
            pl.BlockSpec(memory_space=pltpu.MemorySpace.HBM),
            pl.BlockSpec((_W1ROWS, _HIDDEN), lambda q: (q, 0)),
            pl.BlockSpec((_HIDDEN,), lambda q: (0,)),
        ],
        out_specs=pl.BlockSpec((_HIDDEN, 1), lambda q: (0, 0)),
        out_shape=jax.ShapeDtypeStruct((_HIDDEN, 1), jnp.float32),
        scratch_shapes=[
            pltpu.VMEM((_NPOS, _EMB), jnp.float32),
            pltpu.VMEM((1, _XLEN), jnp.float32),
            pltpu.VMEM((1, _HIDDEN), jnp.float32),
            pltpu.SemaphoreType.DMA,
        ],
    )(idx, emb, W1, b1)


def _lin2_body(hc_ref, w2_ref, b2_ref, o_ref, acc_ref, m_ref, l_ref):
    k = pl.program_id(0)

    @pl.when(k < _NBLK)
    def _():
        zp = (jnp.sum(w2_ref[...] * hc_ref[...], axis=0, keepdims=True)
              + b2_ref[...])
        lane = jax.lax.broadcasted_iota(jnp.int32, (1, _VBLK), 1)
        valid = (k * _VBLK + lane) < _VOCAB
        zm = jnp.where(valid, zp, -jnp.inf)
        mk = jnp.max(zm)
        lk = jnp.sum(jnp.where(valid, jnp.exp(zm - mk), 0.0))
        for j in range(_NBLK):
            @pl.when(k == j)
            def _():
                acc_ref[0:1, pl.ds(j * _VBLK, _VBLK)] = zp

        @pl.when(k == 0)
        def _():
            m_ref[0] = mk
            l_ref[0] = lk

        @pl.when(k > 0)
        def _():
            m_old = m_ref[0]
            m_new = jnp.maximum(m_old, mk)
            l_ref[0] = (l_ref[0] * jnp.exp(m_old - m_new)
                        + lk * jnp.exp(mk - m_new))
            m_ref[0] = m_new

    @pl.when(k == _NBLK)
    def _():
        lse = jnp.log(l_ref[0]) + m_ref[0]
        o_ref[...] = acc_ref[0:1, pl.ds(0, _VOCAB)] - lse


def _lin2_logsoftmax(h_col, W2, b2):
    grid_spec = pltpu.PrefetchScalarGridSpec(
        num_scalar_prefetch=0,
        grid=(_NBLK + 1,),
        in_specs=[
            pl.BlockSpec((_HIDDEN, 1), lambda k: (0, 0)),
            pl.BlockSpec((_HIDDEN, _VBLK),
                         lambda k: (0, jnp.minimum(k, _NBLK - 1))),
            pl.BlockSpec((1, _VBLK),
                         lambda k: (0, jnp.minimum(k, _NBLK - 1))),
        ],
        out_specs=pl.BlockSpec((1, _VOCAB), lambda k: (0, 0)),
        scratch_shapes=[
            pltpu.VMEM((1, _NBLK * _VBLK), jnp.float32),
            pltpu.SMEM((1,), jnp.float32),
            pltpu.SMEM((1,), jnp.float32),
        ],
    )
    return pl.pallas_call(
        _lin2_body,
        grid_spec=grid_spec,
        out_shape=jax.ShapeDtypeStruct((1, _VOCAB), jnp.float32),
    )(h_col, W2, b2.reshape(1, _VOCAB))


def kernel(inp, emb, W1, b1, W2, b2):
    idx = inp.astype(jnp.int32)
    h_col = _lin1(idx, emb, W1, b1)
    return _lin2_logsoftmax(h_col, W2, b2)


# speedup vs baseline: 2.1180x; 1.0340x over previous
import jax
import jax.numpy as jnp
from jax.experimental import pallas as pl
from jax.experimental.pallas import tpu as pltpu

_VOCAB = 100000
_EMB = 128
_NPOS = 200
_HIDDEN = 50
_XLEN = _NPOS * _EMB
_W1BLKS = 2
_W1ROWS = _XLEN // _W1BLKS
_VBLK = 25600
_NBLK = 4


def _lin1_body(idx_ref, emb_hbm, w1_ref, b1_ref, o_ref,
               xbuf, xrow, hacc, gsem):
    q = pl.program_id(0)

    @pl.when(q == 0)
    def _():
        for j in range(_NPOS):
            pltpu.make_async_copy(
                emb_hbm.at[pl.ds(idx_ref[j], 1), :],
                xbuf.at[pl.ds(j, 1), :], gsem).start()
        pltpu.make_async_copy(
            emb_hbm.at[pl.ds(0, _NPOS), :], xbuf, gsem).wait()
        xrow[...] = jnp.reshape(xbuf[...], (1, _XLEN))

    xc = xrow[0:1, pl.ds(q * _W1ROWS, _W1ROWS)].astype(jnp.bfloat16)
    part = jnp.dot(xc, w1_ref[...].astype(jnp.bfloat16),
                   preferred_element_type=jnp.float32)

    @pl.when(q == 0)
    def _():
        hacc[...] = part

    @pl.when(q > 0)
    def _():
        hacc[...] += part

    @pl.when(q == _W1BLKS - 1)
    def _():
        h = jnp.maximum(hacc[...] + b1_ref[...][None, :], 0.0)
        eye = (jax.lax.broadcasted_iota(jnp.int32, (_HIDDEN, _HIDDEN), 0)
               == jax.lax.broadcasted_iota(jnp.int32, (_HIDDEN, _HIDDEN), 1)
               ).astype(jnp.float32)
        o_ref[...] = jnp.sum(
            jnp.dot(jnp.ones((_HIDDEN, 1), jnp.float32), h,
                    preferred_element_type=jnp.float32) * eye,
            axis=1, keepdims=True)


def _lin1(idx, emb, W1, b1):
    return pl.pallas_call(
        _lin1_body,
        grid=(_W1BLKS,),
        in_specs=[
            pl.BlockSpec(memory_space=pltpu.MemorySpace.SMEM),
            pl.BlockSpec(memory_space=pltpu.MemorySpace.HBM),
            pl.BlockSpec((_W1ROWS, _HIDDEN), lambda q: (q, 0)),
            pl.BlockSpec((_HIDDEN,), lambda q: (0,)),
        ],
        out_specs=pl.BlockSpec((_HIDDEN, 1), lambda q: (0, 0)),
        out_shape=jax.ShapeDtypeStruct((_HIDDEN, 1), jnp.float32),
        scratch_shapes=[
            pltpu.VMEM((_NPOS, _EMB), jnp.float32),
            pltpu.VMEM((1, _XLEN), jnp.float32),
            pltpu.VMEM((1, _HIDDEN), jnp.float32),
            pltpu.SemaphoreType.DMA,
        ],
    )(idx, emb, W1, b1)


def _lin2_body(hc_ref, w2_ref, b2_ref, o_ref, acc_ref, m_ref, l_ref):
    k = pl.program_id(0)

    @pl.when(k < _NBLK)
    def _():
        zp = (jnp.sum(w2_ref[...] * hc_ref[...], axis=0, keepdims=True)
              + b2_ref[...])
        lane = jax.lax.broadcasted_iota(jnp.int32, (1, _VBLK), 1)
        valid = (k * _VBLK + lane) < _VOCAB
        zm = jnp.where(valid, zp, -jnp.inf)
        mk = jnp.max(zm)
        lk = jnp.sum(jnp.where(valid, jnp.exp(zm - mk), 0.0))
        for j in range(_NBLK):
            @pl.when(k == j)
            def _():
                acc_ref[0:1, pl.ds(j * _VBLK, _VBLK)] = zp

        @pl.when(k == 0)
        def _():
            m_ref[0] = mk
            l_ref[0] = lk

        @pl.when(k > 0)
        def _():
            m_old = m_ref[0]
            m_new = jnp.maximum(m_old, mk)
            l_ref[0] = (l_ref[0] * jnp.exp(m_old - m_new)
                        + lk * jnp.exp(mk - m_new))
            m_ref[0] = m_new

    @pl.when(k == _NBLK)
    def _():
        lse = jnp.log(l_ref[0]) + m_ref[0]
        o_ref[...] = acc_ref[0:1, pl.ds(0, _VOCAB)] - lse


def _lin2_logsoftmax(h_col, W2, b2):
    grid_spec = pltpu.PrefetchScalarGridSpec(
        num_scalar_prefetch=0,
        grid=(_NBLK + 1,),
        in_specs=[
            pl.BlockSpec((_HIDDEN, 1), lambda k: (0, 0)),
            pl.BlockSpec((_HIDDEN, _VBLK),
                         lambda k: (0, jnp.minimum(k, _NBLK - 1))),
            pl.BlockSpec((1, _VBLK),
                         lambda k: (0, jnp.minimum(k, _NBLK - 1))),
        ],
        out_specs=pl.BlockSpec((1, _VOCAB), lambda k: (0, 0)),
        scratch_shapes=[
            pltpu.VMEM((1, _NBLK * _VBLK), jnp.float32),
            pltpu.SMEM((1,), jnp.float32),
            pltpu.SMEM((1,), jnp.float32),
        ],
    )
    return pl.pallas_call(
        _lin2_body,
        grid_spec=grid_spec,
        out_shape=jax.ShapeDtypeStruct((1, _VOCAB), jnp.float32),
    )(h_col, W2, b2.reshape(1, _VOCAB))


def kernel(inp, emb, W1, b1, W2, b2):
    idx = inp.astype(jnp.int32)
    h_col = _lin1(idx, emb, W1, b1)
    return _lin2_logsoftmax(h_col, W2, b2)


# speedup vs baseline: 2.1380x; 1.0094x over previous
import jax
import jax.numpy as jnp
from jax.experimental import pallas as pl
from jax.experimental.pallas import tpu as pltpu

_VOCAB = 100000
_EMB = 128
_NPOS = 200
_HIDDEN = 50
_XLEN = _NPOS * _EMB
_W1BLKS = 2
_W1ROWS = _XLEN // _W1BLKS
_VBLK = 51200
_NBLK = 2


def _lin1_body(idx_ref, emb_hbm, w1_ref, b1_ref, o_ref,
               xbuf, xrow, hacc, gsem):
    q = pl.program_id(0)

    @pl.when(q == 0)
    def _():
        for j in range(_NPOS):
            pltpu.make_async_copy(
                emb_hbm.at[pl.ds(idx_ref[j], 1), :],
                xbuf.at[pl.ds(j, 1), :], gsem).start()
        pltpu.make_async_copy(
            emb_hbm.at[pl.ds(0, _NPOS), :], xbuf, gsem).wait()
        xrow[...] = jnp.reshape(xbuf[...], (1, _XLEN))

    xc = xrow[0:1, pl.ds(q * _W1ROWS, _W1ROWS)].astype(jnp.bfloat16)
    part = jnp.dot(xc, w1_ref[...].astype(jnp.bfloat16),
                   preferred_element_type=jnp.float32)

    @pl.when(q == 0)
    def _():
        hacc[...] = part

    @pl.when(q > 0)
    def _():
        hacc[...] += part

    @pl.when(q == _W1BLKS - 1)
    def _():
        h = jnp.maximum(hacc[...] + b1_ref[...][None, :], 0.0)
        eye = (jax.lax.broadcasted_iota(jnp.int32, (_HIDDEN, _HIDDEN), 0)
               == jax.lax.broadcasted_iota(jnp.int32, (_HIDDEN, _HIDDEN), 1)
               ).astype(jnp.float32)
        o_ref[...] = jnp.sum(
            jnp.dot(jnp.ones((_HIDDEN, 1), jnp.float32), h,
                    preferred_element_type=jnp.float32) * eye,
            axis=1, keepdims=True)


def _lin1(idx, emb, W1, b1):
    return pl.pallas_call(
        _lin1_body,
        grid=(_W1BLKS,),
        in_specs=[
            pl.BlockSpec(memory_space=pltpu.MemorySpace.SMEM),
            pl.BlockSpec(memory_space=pltpu.MemorySpace.HBM),
            pl.BlockSpec((_W1ROWS, _HIDDEN), lambda q: (q, 0)),
            pl.BlockSpec((_HIDDEN,), lambda q: (0,)),
        ],
        out_specs=pl.BlockSpec((_HIDDEN, 1), lambda q: (0, 0)),
        out_shape=jax.ShapeDtypeStruct((_HIDDEN, 1), jnp.float32),
        scratch_shapes=[
            pltpu.VMEM((_NPOS, _EMB), jnp.float32),
            pltpu.VMEM((1, _XLEN), jnp.float32),
            pltpu.VMEM((1, _HIDDEN), jnp.float32),
            pltpu.SemaphoreType.DMA,
        ],
    )(idx, emb, W1, b1)


def _lin2_body(hc_ref, w2_ref, b2_ref, o_ref, acc_ref, m_ref, l_ref):
    k = pl.program_id(0)

    @pl.when(k < _NBLK)
    def _():
        zp = (jnp.sum(w2_ref[...] * hc_ref[...], axis=0, keepdims=True)
              + b2_ref[...])
        lane = jax.lax.broadcasted_iota(jnp.int32, (1, _VBLK), 1)
        valid = (k * _VBLK + lane) < _VOCAB
        zm = jnp.where(valid, zp, -jnp.inf)
        mk = jnp.max(zm)
        lk = jnp.sum(jnp.where(valid, jnp.exp(zm - mk), 0.0))
        for j in range(_NBLK):
            @pl.when(k == j)
            def _():
                acc_ref[0:1, pl.ds(j * _VBLK, _VBLK)] = zp

        @pl.when(k == 0)
        def _():
            m_ref[0] = mk
            l_ref[0] = lk

        @pl.when(k > 0)
        def _():
            m_old = m_ref[0]
            m_new = jnp.maximum(m_old, mk)
            l_ref[0] = (l_ref[0] * jnp.exp(m_old - m_new)
                        + lk * jnp.exp(mk - m_new))
            m_ref[0] = m_new

    @pl.when(k == _NBLK)
    def _():
        lse = jnp.log(l_ref[0]) + m_ref[0]
        o_ref[...] = acc_ref[0:1, pl.ds(0, _VOCAB)] - lse


def _lin2_logsoftmax(h_col, W2, b2):
    grid_spec = pltpu.PrefetchScalarGridSpec(
        num_scalar_prefetch=0,
        grid=(_NBLK + 1,),
        in_specs=[
            pl.BlockSpec((_HIDDEN, 1), lambda k: (0, 0)),
            pl.BlockSpec((_HIDDEN, _VBLK),
                         lambda k: (0, jnp.minimum(k, _NBLK - 1))),
            pl.BlockSpec((1, _VBLK),
                         lambda k: (0, jnp.minimum(k, _NBLK - 1))),
        ],
        out_specs=pl.BlockSpec((1, _VOCAB), lambda k: (0, 0)),
        scratch_shapes=[
            pltpu.VMEM((1, _NBLK * _VBLK), jnp.float32),
            pltpu.SMEM((1,), jnp.float32),
            pltpu.SMEM((1,), jnp.float32),
        ],
    )
    return pl.pallas_call(
        _lin2_body,
        grid_spec=grid_spec,
        out_shape=jax.ShapeDtypeStruct((1, _VOCAB), jnp.float32),
    )(h_col, W2, b2.reshape(1, _VOCAB))


def kernel(inp, emb, W1, b1, W2, b2):
    idx = inp.astype(jnp.int32)
    h_col = _lin1(idx, emb, W1, b1)
    return _lin2_logsoftmax(h_col, W2, b2)
